# baseline (speedup 1.0000x reference)
import functools
import math

import jax
import jax.numpy as jnp
from jax import lax
from jax.experimental import pallas as pl
from jax.experimental.pallas import tpu as pltpu
from jax.experimental.pallas import tpu_sc as plsc

N_NODES = 50000
N_SESS = 1024
D = 64
V = 100000
BV = 2048
NBLK = (V + BV - 1) // BV
NPADCOL = NBLK * BV - V

N_PAD = 51200
NW = 32
R_NODE = N_PAD // NW
R_LAST = N_SESS // NW
INV_SQRT2 = 1.0 / math.sqrt(2.0)

_SC_MESH = dict(core_axis_name="c", subcore_axis_name="s")


DP = 128
RC = R_NODE // 2


BT = 8192
NBT = (V + BT - 1) // BT


def _norm_t_body(embt_ref, out_ref):
    x = embt_ref[...]
    n = jnp.sqrt(jnp.sum(x * x, axis=0, keepdims=True))
    out_ref[:, 0:D] = (x / (n + 1e-12)).T


def _target_rows(embt):
    return pl.pallas_call(
        _norm_t_body,
        grid=(NBT,),
        in_specs=[pl.BlockSpec((D, BT), lambda j: (0, j))],
        out_specs=pl.BlockSpec((BT, DP), lambda j: (j, 0)),
        out_shape=jax.ShapeDtypeStruct((V, DP), jnp.float32),
        compiler_params=pltpu.CompilerParams(
            dimension_semantics=("arbitrary",)),
    )(embt)


NCH = 8
NBUF = 4
RQ = R_NODE // NCH


def _sc_gather_rows(table, idx):
    mesh = plsc.VectorSubcoreMesh(**_SC_MESH)

    @functools.partial(
        pl.kernel, mesh=mesh,
        out_type=jax.ShapeDtypeStruct((N_PAD, DP), jnp.float32),
        scratch_types=(
            [pltpu.VMEM((RQ,), jnp.int32)] * NBUF
            + [pltpu.VMEM((RQ, DP), jnp.float32)] * NBUF
            + [pltpu.SemaphoreType.DMA] * NBUF
        ),
    )
    def k(tbl_hbm, idx_hbm, out_hbm, *scr):
        idx_v = scr[0:NBUF]
        rows_v = scr[NBUF:2 * NBUF]
        sems = scr[2 * NBUF:3 * NBUF]
        wid = lax.axis_index("s") * 2 + lax.axis_index("c")
        base = wid * R_NODE
        handles = [None] * NBUF
        for c in range(NCH):
            b = c % NBUF
            if handles[b] is not None:
                handles[b].wait()
                pltpu.sync_copy(rows_v[b],
                                out_hbm.at[pl.ds(base + (c - NBUF) * RQ, RQ)])
            pltpu.sync_copy(idx_hbm.at[pl.ds(base + c * RQ, RQ)], idx_v[b])
            handles[b] = pltpu.async_copy(tbl_hbm.at[idx_v[b]], rows_v[b],
                                          sems[b])
        for c in range(NCH - NBUF, NCH):
            b = c % NBUF
            handles[b].wait()
            pltpu.sync_copy(rows_v[b],
                            out_hbm.at[pl.ds(base + c * RQ, RQ)])

    return k(table, idx)


def _sc_gather_last(feat, last_nodes):
    mesh = plsc.VectorSubcoreMesh(**_SC_MESH)

    @functools.partial(
        pl.kernel, mesh=mesh,
        out_type=jax.ShapeDtypeStruct((N_SESS, DP), jnp.float32),
        scratch_types=[
            pltpu.VMEM((R_LAST,), jnp.int32),
            pltpu.VMEM((R_LAST, DP), jnp.float32),
            pltpu.SemaphoreType.DMA,
        ],
    )
    def k(feat_hbm, last_hbm, lrows_hbm, idx_v, rows_v, sem):
        wid = lax.axis_index("s") * 2 + lax.axis_index("c")
        base = wid * R_LAST
        pltpu.sync_copy(last_hbm.at[pl.ds(base, R_LAST)], idx_v)
        pltpu.async_copy(feat_hbm.at[idx_v], rows_v, sem).wait()
        pltpu.sync_copy(rows_v, lrows_hbm.at[pl.ds(base, R_LAST)])

    return k(feat, last_nodes)


BP = N_SESS + 8


def _featv_body(lrows_ref, wv_ref, b_ref, out_ref):
    gl = lrows_ref[...][:, :D] * INV_SQRT2
    wv = wv_ref[0:D, :] + wv_ref[D:2 * D, :]
    fv = jax.lax.dot_general(gl, wv, (((1,), (0,)), ((), ())),
                             preferred_element_type=jnp.float32)
    fv = fv + b_ref[...]
    out_ref[...] = jnp.concatenate(
        [fv.T, jnp.zeros((D, BP - N_SESS), jnp.float32)], axis=1)


def _featv(lrows, fc_v_w, fc_v_b):
    return pl.pallas_call(
        _featv_body,
        out_shape=jax.ShapeDtypeStruct((D, BP), jnp.float32),
    )(lrows, fc_v_w, fc_v_b.reshape(1, D))


BN = 2048
NBN = N_PAD // BN


def _xe_body(feat_ref, segrow_ref, fvt_ref, wu_ref, we_ref, xe_ref, w_ref):
    feat = feat_ref[...]
    f = feat[:, :D]
    wu = (wu_ref[0:D, :] + wu_ref[D:2 * D, :]) * INV_SQRT2
    u = jax.lax.dot_general(f, wu, (((1,), (0,)), ((), ())),
                            preferred_element_type=jnp.float32)
    seg = segrow_ref[...]
    sidx = jax.lax.broadcasted_iota(jnp.int32, (BP, BN), 0)
    onehot = jnp.where(sidx == seg, 1.0, 0.0)
    vt = jax.lax.dot_general(fvt_ref[...], onehot, (((1,), (0,)), ((), ())),
                             preferred_element_type=jnp.float32)
    x = u + vt.T
    sg = 1.0 / (1.0 + jnp.exp(-x))
    e = jax.lax.dot_general(sg, we_ref[...], (((1,), (0,)), ((), ())),
                            preferred_element_type=jnp.float32)
    xe = jnp.exp(e)
    xe_ref[...] = xe.reshape(BN // 128, 128)
    w_ref[...] = feat * xe


def _xe(feat, seg_row, fv_t, fc_u_w, fc_e_w):
    return pl.pallas_call(
        _xe_body,
        grid=(NBN,),
        in_specs=[
            pl.BlockSpec((BN, DP), lambda j: (j, 0)),
            pl.BlockSpec((1, BN), lambda j: (0, j)),
            pl.BlockSpec((D, BP), lambda j: (0, 0)),
            pl.BlockSpec((2 * D, D), lambda j: (0, 0)),
            pl.BlockSpec((D, 1), lambda j: (0, 0)),
        ],
        out_specs=[
            pl.BlockSpec((BN // 128, 128), lambda j: (j, 0)),
            pl.BlockSpec((BN, DP), lambda j: (j, 0)),
        ],
        out_shape=[
            jax.ShapeDtypeStruct((N_PAD // 128, 128), jnp.float32),
            jax.ShapeDtypeStruct((N_PAD, DP), jnp.float32),
        ],
        compiler_params=pltpu.CompilerParams(
            dimension_semantics=("arbitrary",)),
    )(feat, seg_row, fv_t, fc_u_w, fc_e_w)


SEGP = 1040
NV16 = R_NODE // 16
NS16 = SEGP // 16


def _sc_segsum(xe_flat, seg_pad):
    mesh = plsc.VectorSubcoreMesh(**_SC_MESH)

    @functools.partial(
        pl.kernel, mesh=mesh,
        out_type=jax.ShapeDtypeStruct((2, SEGP), jnp.float32),
        scratch_types=[
            pltpu.VMEM((R_NODE,), jnp.float32),
            pltpu.VMEM((R_NODE,), jnp.int32),
            pltpu.VMEM((SEGP,), jnp.float32),
            pltpu.VMEM((16 * SEGP,), jnp.float32),
            pltpu.VMEM_SHARED((16 * SEGP,), jnp.float32),
            pltpu.SemaphoreType.DMA,
        ],
        compiler_params=pltpu.CompilerParams(needs_layout_passes=False),
    )
    def k(xe_hbm, seg_hbm, parts_hbm, xev, segv, local, gbuf, shared, sem):
        cid = lax.axis_index("c")
        sid = lax.axis_index("s")
        wid = sid * 2 + cid
        base = wid * R_NODE
        pltpu.sync_copy(xe_hbm.at[pl.ds(base, R_NODE)], xev)
        pltpu.sync_copy(seg_hbm.at[pl.ds(base, R_NODE)], segv)

        def zero_body(i, _):
            local[pl.ds(i * 16, 16)] = jnp.zeros((16,), jnp.float32)
            return 0

        lax.fori_loop(0, NS16, zero_body, 0)

        def acc_body(i, _):
            sg = segv[pl.ds(i * 16, 16)]
            xv = xev[pl.ds(i * 16, 16)]
            plsc.addupdate_scatter(local, [sg], xv)
            return 0

        lax.fori_loop(0, NV16, acc_body, 0)
        pltpu.sync_copy(local, shared.at[pl.ds(sid * SEGP, SEGP)])
        plsc.subcore_barrier()
        pltpu.sync_copy(shared, gbuf)

        def red_body(j, _):
            acc = jnp.zeros((16,), jnp.float32)
            for i in range(16):
                acc = acc + gbuf[pl.ds(i * SEGP + j * 16, 16)]
            local[pl.ds(j * 16, 16)] = acc
            return 0

        lax.fori_loop(0, NS16, red_body, 0)

        @pl.when(sid == 0)
        def _():
            pltpu.sync_copy(local, parts_hbm.at[cid])

    return k(xe_flat, seg_pad)


def _sc_scatter_h(w, seg_pad, zeros_h):
    mesh = plsc.VectorSubcoreMesh(**_SC_MESH)

    @functools.partial(
        pl.kernel, mesh=mesh,
        out_type=jax.ShapeDtypeStruct((2, BP, DP), jnp.float32),
        scratch_types=[
            pltpu.VMEM((RC,), jnp.int32),
            pltpu.VMEM((RC, DP), jnp.float32),
            pltpu.VMEM_SHARED((BP, DP), jnp.float32),
            pltpu.SemaphoreType.DMA,
        ],
        compiler_params=pltpu.CompilerParams(needs_layout_passes=False),
    )
    def k(w_hbm, seg_hbm, zeros_hbm, h_out, segv, fbuf, h_sh, sem):
        cid = lax.axis_index("c")
        sid = lax.axis_index("s")
        wid = sid * 2 + cid

        @pl.when(sid == 0)
        def _():
            pltpu.sync_copy(zeros_hbm, h_sh)

        plsc.subcore_barrier()
        for half in range(2):
            base = wid * R_NODE + half * RC
            pltpu.sync_copy(w_hbm.at[pl.ds(base, RC)], fbuf)
            pltpu.sync_copy(seg_hbm.at[pl.ds(base, RC)], segv)
            pltpu.sync_copy(fbuf, h_sh.at[segv], add=True)
        plsc.subcore_barrier()

        @pl.when(sid == 0)
        def _():
            pltpu.sync_copy(h_sh, h_out.at[cid])

    return k(w, seg_pad, zeros_h)


def _sr_body(hp_ref, parts_ref, lrows_ref, wo_ref, ws_ref, out_ref):
    s_row = parts_ref[0:1, 0:N_SESS] + parts_ref[1:2, 0:N_SESS]
    s_col = jax.lax.transpose(s_row, (1, 0))
    h = (hp_ref[0][:N_SESS, :D] + hp_ref[1][:N_SESS, :D]) * (
        INV_SQRT2 / jnp.maximum(s_col, 1e-30))
    gl = lrows_ref[...][:, :D] * INV_SQRT2
    wo = wo_ref[0:D, :] + wo_ref[D:2 * D, :]
    sr_g = jax.lax.dot_general(h, wo, (((1,), (0,)), ((), ())),
                               preferred_element_type=jnp.float32)
    ws12 = ws_ref[0:D, :] + ws_ref[D:2 * D, :]
    sr = (jax.lax.dot_general(gl, ws12, (((1,), (0,)), ((), ())),
                              preferred_element_type=jnp.float32)
          + jax.lax.dot_general(sr_g, ws_ref[2 * D:3 * D, :],
                                (((1,), (0,)), ((), ())),
                                preferred_element_type=jnp.float32))
    n = jnp.sqrt(jnp.sum(sr * sr, axis=1, keepdims=True))
    out_ref[...] = sr / (n + 1e-12)


def _sr(h_parts, parts, lrows, fc_out_w, fc_sr_w):
    return pl.pallas_call(
        _sr_body,
        out_shape=jax.ShapeDtypeStruct((N_SESS, D), jnp.float32),
    )(h_parts, parts, lrows, fc_out_w, fc_sr_w)


def _lse_body(sr_ref, embt_ref, lse_ref, s_scr):
    j = pl.program_id(0)

    @pl.when(j == 0)
    def _():
        s_scr[...] = jnp.zeros_like(s_scr)

    embt = embt_ref[...]
    nrm = jnp.sqrt(jnp.sum(embt * embt, axis=0, keepdims=True))
    col = j * BV + jax.lax.broadcasted_iota(jnp.int32, (1, BV), 1)
    t = jnp.where(col < V, embt / (nrm + 1e-12), 0.0)
    sr = sr_ref[...]
    logits_t = jax.lax.dot_general(
        t.astype(jnp.bfloat16), sr.astype(jnp.bfloat16),
        (((0,), (1,)), ((), ())), preferred_element_type=jnp.float32)
    s_new = s_scr[...] + jnp.sum(jnp.exp(logits_t), axis=0, keepdims=True)
    s_scr[...] = s_new

    @pl.when(j == NBLK - 1)
    def _():
        lse_ref[...] = jnp.log(s_new - float(NPADCOL))


def _out_body(sr_ref, lse_ref, embt_ref, out_ref):
    embt = embt_ref[...]
    nrm = jnp.sqrt(jnp.sum(embt * embt, axis=0, keepdims=True))
    t = embt / (nrm + 1e-12)
    sr = sr_ref[...]
    logits_t = jax.lax.dot_general(
        t.astype(jnp.bfloat16), sr.astype(jnp.bfloat16),
        (((0,), (1,)), ((), ())), preferred_element_type=jnp.float32)
    out_ref[...] = logits_t - lse_ref[...]


def _phase_b(sr, embt):
    lse = pl.pallas_call(
        _lse_body,
        grid=(NBLK,),
        in_specs=[
            pl.BlockSpec((N_SESS, D), lambda j: (0, 0)),
            pl.BlockSpec((D, BV), lambda j: (0, j)),
        ],
        out_specs=pl.BlockSpec((1, N_SESS), lambda j: (0, 0)),
        out_shape=jax.ShapeDtypeStruct((1, N_SESS), jnp.float32),
        scratch_shapes=[pltpu.VMEM((1, N_SESS), jnp.float32)],
        compiler_params=pltpu.CompilerParams(
            dimension_semantics=("arbitrary",)),
    )(sr, embt)
    out_t = pl.pallas_call(
        _out_body,
        grid=(NBLK,),
        in_specs=[
            pl.BlockSpec((N_SESS, D), lambda j: (0, 0)),
            pl.BlockSpec((1, N_SESS), lambda j: (0, 0)),
            pl.BlockSpec((D, BV), lambda j: (0, j)),
        ],
        out_specs=pl.BlockSpec((BV, N_SESS), lambda j: (j, 0)),
        out_shape=jax.ShapeDtypeStruct((V, N_SESS), jnp.float32),
        compiler_params=pltpu.CompilerParams(
            dimension_semantics=("arbitrary",)),
    )(sr, lse, embt)
    return out_t


def kernel(iid, last_nodes, segment_ids, embedding, fc_u_w, fc_v_w, fc_v_b,
           fc_e_w, fc_out_w, fc_sr_w):
    embt = embedding.T
    target = _target_rows(embt)
    iid_pad = jnp.pad(iid, (0, N_PAD - N_NODES))
    seg_pad = jnp.pad(segment_ids, (0, N_PAD - N_NODES),
                      constant_values=N_SESS)
    feat = _sc_gather_rows(target, iid_pad)
    lrows = _sc_gather_last(feat, last_nodes)
    fv_t = _featv(lrows, fc_v_w, fc_v_b)
    xe2d, w = _xe(feat, seg_pad.reshape(1, N_PAD), fv_t, fc_u_w, fc_e_w)
    xe_flat = xe2d.reshape(N_PAD)
    parts = _sc_segsum(xe_flat, seg_pad)
    zeros_h = jnp.zeros((BP, DP), jnp.float32)
    h_parts = _sc_scatter_h(w, seg_pad, zeros_h)
    sr = _sr(h_parts, parts, lrows, fc_out_w, fc_sr_w)
    return _phase_b(sr, embt).T


# speedup vs baseline: 1.0176x; 1.0176x over previous
import functools
import math

import jax
import jax.numpy as jnp
from jax import lax
from jax.experimental import pallas as pl
from jax.experimental.pallas import tpu as pltpu
from jax.experimental.pallas import tpu_sc as plsc

N_NODES = 50000
N_SESS = 1024
D = 64
V = 100000
BV = 4096
NBLK = (V + BV - 1) // BV
NPADCOL = NBLK * BV - V

N_PAD = 51200
NW = 32
R_NODE = N_PAD // NW
R_LAST = N_SESS // NW
INV_SQRT2 = 1.0 / math.sqrt(2.0)

_SC_MESH = dict(core_axis_name="c", subcore_axis_name="s")


DP = 128
RC = R_NODE // 2


BT = 8192
NBT = (V + BT - 1) // BT


def _norm_t_body(embt_ref, out_ref):
    x = embt_ref[...]
    n = jnp.sqrt(jnp.sum(x * x, axis=0, keepdims=True))
    out_ref[:, 0:D] = (x / (n + 1e-12)).T


def _target_rows(embt):
    return pl.pallas_call(
        _norm_t_body,
        grid=(NBT,),
        in_specs=[pl.BlockSpec((D, BT), lambda j: (0, j))],
        out_specs=pl.BlockSpec((BT, DP), lambda j: (j, 0)),
        out_shape=jax.ShapeDtypeStruct((V, DP), jnp.float32),
        compiler_params=pltpu.CompilerParams(
            dimension_semantics=("arbitrary",)),
    )(embt)


NCH = 8
NBUF = 4
RQ = R_NODE // NCH


def _sc_gather_rows(table, idx):
    mesh = plsc.VectorSubcoreMesh(**_SC_MESH)

    @functools.partial(
        pl.kernel, mesh=mesh,
        out_type=jax.ShapeDtypeStruct((N_PAD, DP), jnp.float32),
        scratch_types=(
            [pltpu.VMEM((RQ,), jnp.int32)] * NBUF
            + [pltpu.VMEM((RQ, DP), jnp.float32)] * NBUF
            + [pltpu.SemaphoreType.DMA] * NBUF
        ),
    )
    def k(tbl_hbm, idx_hbm, out_hbm, *scr):
        idx_v = scr[0:NBUF]
        rows_v = scr[NBUF:2 * NBUF]
        sems = scr[2 * NBUF:3 * NBUF]
        wid = lax.axis_index("s") * 2 + lax.axis_index("c")
        base = wid * R_NODE
        handles = [None] * NBUF
        for c in range(NCH):
            b = c % NBUF
            if handles[b] is not None:
                handles[b].wait()
                pltpu.sync_copy(rows_v[b],
                                out_hbm.at[pl.ds(base + (c - NBUF) * RQ, RQ)])
            pltpu.sync_copy(idx_hbm.at[pl.ds(base + c * RQ, RQ)], idx_v[b])
            handles[b] = pltpu.async_copy(tbl_hbm.at[idx_v[b]], rows_v[b],
                                          sems[b])
        for c in range(NCH - NBUF, NCH):
            b = c % NBUF
            handles[b].wait()
            pltpu.sync_copy(rows_v[b],
                            out_hbm.at[pl.ds(base + c * RQ, RQ)])

    return k(table, idx)


def _sc_gather_last(feat, last_nodes):
    mesh = plsc.VectorSubcoreMesh(**_SC_MESH)

    @functools.partial(
        pl.kernel, mesh=mesh,
        out_type=jax.ShapeDtypeStruct((N_SESS, DP), jnp.float32),
        scratch_types=[
            pltpu.VMEM((R_LAST,), jnp.int32),
            pltpu.VMEM((R_LAST, DP), jnp.float32),
            pltpu.SemaphoreType.DMA,
        ],
    )
    def k(feat_hbm, last_hbm, lrows_hbm, idx_v, rows_v, sem):
        wid = lax.axis_index("s") * 2 + lax.axis_index("c")
        base = wid * R_LAST
        pltpu.sync_copy(last_hbm.at[pl.ds(base, R_LAST)], idx_v)
        pltpu.async_copy(feat_hbm.at[idx_v], rows_v, sem).wait()
        pltpu.sync_copy(rows_v, lrows_hbm.at[pl.ds(base, R_LAST)])

    return k(feat, last_nodes)


BP = N_SESS + 8


def _featv_body(lrows_ref, wv_ref, b_ref, out_ref):
    gl = lrows_ref[...][:, :D] * INV_SQRT2
    wv = wv_ref[0:D, :] + wv_ref[D:2 * D, :]
    fv = jax.lax.dot_general(gl, wv, (((1,), (0,)), ((), ())),
                             preferred_element_type=jnp.float32)
    fv = fv + b_ref[...]
    out_ref[...] = jnp.concatenate(
        [fv.T, jnp.zeros((D, BP - N_SESS), jnp.float32)], axis=1)


def _featv(lrows, fc_v_w, fc_v_b):
    return pl.pallas_call(
        _featv_body,
        out_shape=jax.ShapeDtypeStruct((D, BP), jnp.float32),
    )(lrows, fc_v_w, fc_v_b.reshape(1, D))


BN = 2048
NBN = N_PAD // BN


def _xe_body(feat_ref, segrow_ref, fvt_ref, wu_ref, we_ref, xe_ref, w_ref):
    feat = feat_ref[...]
    f = feat[:, :D]
    wu = (wu_ref[0:D, :] + wu_ref[D:2 * D, :]) * INV_SQRT2
    u = jax.lax.dot_general(f, wu, (((1,), (0,)), ((), ())),
                            preferred_element_type=jnp.float32)
    seg = segrow_ref[...]
    sidx = jax.lax.broadcasted_iota(jnp.int32, (BP, BN), 0)
    onehot = jnp.where(sidx == seg, 1.0, 0.0)
    vt = jax.lax.dot_general(fvt_ref[...], onehot, (((1,), (0,)), ((), ())),
                             preferred_element_type=jnp.float32)
    x = u + vt.T
    sg = 1.0 / (1.0 + jnp.exp(-x))
    e = jax.lax.dot_general(sg, we_ref[...], (((1,), (0,)), ((), ())),
                            preferred_element_type=jnp.float32)
    xe = jnp.exp(e)
    xe_ref[...] = xe.reshape(BN // 128, 128)
    w_ref[...] = feat * xe


def _xe(feat, seg_row, fv_t, fc_u_w, fc_e_w):
    return pl.pallas_call(
        _xe_body,
        grid=(NBN,),
        in_specs=[
            pl.BlockSpec((BN, DP), lambda j: (j, 0)),
            pl.BlockSpec((1, BN), lambda j: (0, j)),
            pl.BlockSpec((D, BP), lambda j: (0, 0)),
            pl.BlockSpec((2 * D, D), lambda j: (0, 0)),
            pl.BlockSpec((D, 1), lambda j: (0, 0)),
        ],
        out_specs=[
            pl.BlockSpec((BN // 128, 128), lambda j: (j, 0)),
            pl.BlockSpec((BN, DP), lambda j: (j, 0)),
        ],
        out_shape=[
            jax.ShapeDtypeStruct((N_PAD // 128, 128), jnp.float32),
            jax.ShapeDtypeStruct((N_PAD, DP), jnp.float32),
        ],
        compiler_params=pltpu.CompilerParams(
            dimension_semantics=("arbitrary",)),
    )(feat, seg_row, fv_t, fc_u_w, fc_e_w)


SEGP = 1040
NV16 = R_NODE // 16
NS16 = SEGP // 16


def _sc_segsum(xe_flat, seg_pad):
    mesh = plsc.VectorSubcoreMesh(**_SC_MESH)

    @functools.partial(
        pl.kernel, mesh=mesh,
        out_type=jax.ShapeDtypeStruct((2, SEGP), jnp.float32),
        scratch_types=[
            pltpu.VMEM((R_NODE,), jnp.float32),
            pltpu.VMEM((R_NODE,), jnp.int32),
            pltpu.VMEM((SEGP,), jnp.float32),
            pltpu.VMEM((16 * SEGP,), jnp.float32),
            pltpu.VMEM_SHARED((16 * SEGP,), jnp.float32),
            pltpu.SemaphoreType.DMA,
        ],
        compiler_params=pltpu.CompilerParams(needs_layout_passes=False),
    )
    def k(xe_hbm, seg_hbm, parts_hbm, xev, segv, local, gbuf, shared, sem):
        cid = lax.axis_index("c")
        sid = lax.axis_index("s")
        wid = sid * 2 + cid
        base = wid * R_NODE
        pltpu.sync_copy(xe_hbm.at[pl.ds(base, R_NODE)], xev)
        pltpu.sync_copy(seg_hbm.at[pl.ds(base, R_NODE)], segv)

        def zero_body(i, _):
            local[pl.ds(i * 16, 16)] = jnp.zeros((16,), jnp.float32)
            return 0

        lax.fori_loop(0, NS16, zero_body, 0)

        def acc_body(i, _):
            sg = segv[pl.ds(i * 16, 16)]
            xv = xev[pl.ds(i * 16, 16)]
            plsc.addupdate_scatter(local, [sg], xv)
            return 0

        lax.fori_loop(0, NV16, acc_body, 0)
        pltpu.sync_copy(local, shared.at[pl.ds(sid * SEGP, SEGP)])
        plsc.subcore_barrier()
        pltpu.sync_copy(shared, gbuf)

        def red_body(j, _):
            acc = jnp.zeros((16,), jnp.float32)
            for i in range(16):
                acc = acc + gbuf[pl.ds(i * SEGP + j * 16, 16)]
            local[pl.ds(j * 16, 16)] = acc
            return 0

        lax.fori_loop(0, NS16, red_body, 0)

        @pl.when(sid == 0)
        def _():
            pltpu.sync_copy(local, parts_hbm.at[cid])

    return k(xe_flat, seg_pad)


def _sc_scatter_h(w, seg_pad, zeros_h):
    mesh = plsc.VectorSubcoreMesh(**_SC_MESH)

    @functools.partial(
        pl.kernel, mesh=mesh,
        out_type=jax.ShapeDtypeStruct((2, BP, DP), jnp.float32),
        scratch_types=[
            pltpu.VMEM((RC,), jnp.int32),
            pltpu.VMEM((RC, DP), jnp.float32),
            pltpu.VMEM_SHARED((BP, DP), jnp.float32),
            pltpu.SemaphoreType.DMA,
        ],
        compiler_params=pltpu.CompilerParams(needs_layout_passes=False),
    )
    def k(w_hbm, seg_hbm, zeros_hbm, h_out, segv, fbuf, h_sh, sem):
        cid = lax.axis_index("c")
        sid = lax.axis_index("s")
        wid = sid * 2 + cid

        @pl.when(sid == 0)
        def _():
            pltpu.sync_copy(zeros_hbm, h_sh)

        plsc.subcore_barrier()
        for half in range(2):
            base = wid * R_NODE + half * RC
            pltpu.sync_copy(w_hbm.at[pl.ds(base, RC)], fbuf)
            pltpu.sync_copy(seg_hbm.at[pl.ds(base, RC)], segv)
            pltpu.sync_copy(fbuf, h_sh.at[segv], add=True)
        plsc.subcore_barrier()

        @pl.when(sid == 0)
        def _():
            pltpu.sync_copy(h_sh, h_out.at[cid])

    return k(w, seg_pad, zeros_h)


def _sr_body(hp_ref, parts_ref, lrows_ref, wo_ref, ws_ref, out_ref):
    s_row = parts_ref[0:1, 0:N_SESS] + parts_ref[1:2, 0:N_SESS]
    s_col = jax.lax.transpose(s_row, (1, 0))
    h = (hp_ref[0][:N_SESS, :D] + hp_ref[1][:N_SESS, :D]) * (
        INV_SQRT2 / jnp.maximum(s_col, 1e-30))
    gl = lrows_ref[...][:, :D] * INV_SQRT2
    wo = wo_ref[0:D, :] + wo_ref[D:2 * D, :]
    sr_g = jax.lax.dot_general(h, wo, (((1,), (0,)), ((), ())),
                               preferred_element_type=jnp.float32)
    ws12 = ws_ref[0:D, :] + ws_ref[D:2 * D, :]
    sr = (jax.lax.dot_general(gl, ws12, (((1,), (0,)), ((), ())),
                              preferred_element_type=jnp.float32)
          + jax.lax.dot_general(sr_g, ws_ref[2 * D:3 * D, :],
                                (((1,), (0,)), ((), ())),
                                preferred_element_type=jnp.float32))
    n = jnp.sqrt(jnp.sum(sr * sr, axis=1, keepdims=True))
    out_ref[...] = sr / (n + 1e-12)


def _sr(h_parts, parts, lrows, fc_out_w, fc_sr_w):
    return pl.pallas_call(
        _sr_body,
        out_shape=jax.ShapeDtypeStruct((N_SESS, D), jnp.float32),
    )(h_parts, parts, lrows, fc_out_w, fc_sr_w)


def _lse_body(sr_ref, embt_ref, lse_ref, s_scr):
    j = pl.program_id(0)

    @pl.when(j == 0)
    def _():
        s_scr[...] = jnp.zeros_like(s_scr)

    embt = embt_ref[...]
    nrm = jnp.sqrt(jnp.sum(embt * embt, axis=0, keepdims=True))
    col = j * BV + jax.lax.broadcasted_iota(jnp.int32, (1, BV), 1)
    t = jnp.where(col < V, embt / (nrm + 1e-12), 0.0)
    sr = sr_ref[...]
    logits_t = jax.lax.dot_general(
        t.astype(jnp.bfloat16), sr.astype(jnp.bfloat16),
        (((0,), (1,)), ((), ())), preferred_element_type=jnp.float32)
    s_new = s_scr[...] + jnp.sum(jnp.exp(logits_t), axis=0, keepdims=True)
    s_scr[...] = s_new

    @pl.when(j == NBLK - 1)
    def _():
        lse_ref[...] = jnp.log(s_new - float(NPADCOL))


def _out_body(sr_ref, lse_ref, embt_ref, out_ref):
    embt = embt_ref[...]
    nrm = jnp.sqrt(jnp.sum(embt * embt, axis=0, keepdims=True))
    t = embt / (nrm + 1e-12)
    sr = sr_ref[...]
    logits_t = jax.lax.dot_general(
        t.astype(jnp.bfloat16), sr.astype(jnp.bfloat16),
        (((0,), (1,)), ((), ())), preferred_element_type=jnp.float32)
    out_ref[...] = logits_t - lse_ref[...]


def _phase_b(sr, embt):
    lse = pl.pallas_call(
        _lse_body,
        grid=(NBLK,),
        in_specs=[
            pl.BlockSpec((N_SESS, D), lambda j: (0, 0)),
            pl.BlockSpec((D, BV), lambda j: (0, j)),
        ],
        out_specs=pl.BlockSpec((1, N_SESS), lambda j: (0, 0)),
        out_shape=jax.ShapeDtypeStruct((1, N_SESS), jnp.float32),
        scratch_shapes=[pltpu.VMEM((1, N_SESS), jnp.float32)],
        compiler_params=pltpu.CompilerParams(
            dimension_semantics=("arbitrary",)),
    )(sr, embt)
    out_t = pl.pallas_call(
        _out_body,
        grid=(NBLK,),
        in_specs=[
            pl.BlockSpec((N_SESS, D), lambda j: (0, 0)),
            pl.BlockSpec((1, N_SESS), lambda j: (0, 0)),
            pl.BlockSpec((D, BV), lambda j: (0, j)),
        ],
        out_specs=pl.BlockSpec((BV, N_SESS), lambda j: (j, 0)),
        out_shape=jax.ShapeDtypeStruct((V, N_SESS), jnp.float32),
        compiler_params=pltpu.CompilerParams(
            dimension_semantics=("arbitrary",)),
    )(sr, lse, embt)
    return out_t


def kernel(iid, last_nodes, segment_ids, embedding, fc_u_w, fc_v_w, fc_v_b,
           fc_e_w, fc_out_w, fc_sr_w):
    embt = embedding.T
    target = _target_rows(embt)
    iid_pad = jnp.pad(iid, (0, N_PAD - N_NODES))
    seg_pad = jnp.pad(segment_ids, (0, N_PAD - N_NODES),
                      constant_values=N_SESS)
    feat = _sc_gather_rows(target, iid_pad)
    lrows = _sc_gather_last(feat, last_nodes)
    fv_t = _featv(lrows, fc_v_w, fc_v_b)
    xe2d, w = _xe(feat, seg_pad.reshape(1, N_PAD), fv_t, fc_u_w, fc_e_w)
    xe_flat = xe2d.reshape(N_PAD)
    parts = _sc_segsum(xe_flat, seg_pad)
    zeros_h = jnp.zeros((BP, DP), jnp.float32)
    h_parts = _sc_scatter_h(w, seg_pad, zeros_h)
    sr = _sr(h_parts, parts, lrows, fc_out_w, fc_sr_w)
    return _phase_b(sr, embt).T
